# baseline (speedup 1.0000x reference)
import jax
import jax.numpy as jnp
from jax import lax
from jax.experimental import pallas as pl
from jax.experimental.pallas import tpu as pltpu
from jax.experimental.pallas import tpu_sc as plsc

N = 10000
E = 320000
D = 128
H = 256
G = 64
HC = 128
NP = 10240
EP = 327680
ER = EP // 128
NC = 2
NS = 16
L = 16
RPT = NP // NS
ERT = ER // NS
CH = 16
NGRP = ERT
PAD_SRC = 10016
PAD_DST = 10008
NB = 16
BR = NP // NB

_sc_mesh = plsc.VectorSubcoreMesh(
    core_axis_name="c", subcore_axis_name="s", num_cores=NC, num_subcores=NS)


def _deg_body(dst_hbm, out_hbm, dacc, dbuf, acc_s, tbuf, sdeg):
    c = lax.axis_index("c")
    s = lax.axis_index("s")
    zero16 = jnp.zeros((L,), jnp.float32)
    ones16 = jnp.ones((L,), jnp.float32)

    def zloop(i, _):
        dacc[pl.ds(i * L, L)] = zero16
        return 0
    lax.fori_loop(0, NP // L, zloop, 0)

    epc = EP // NC
    ept = epc // NS
    base = c * epc + s * ept

    def outer(g, _):
        pltpu.sync_copy(dst_hbm.at[pl.ds(base + g * 2048, 2048)], dbuf)

        def inner(j, _):
            idx = dbuf[pl.ds(j * L, L)]
            plsc.addupdate_scatter(dacc, [idx], ones16)
            return 0
        lax.fori_loop(0, 2048 // L, inner, 0)
        return 0
    lax.fori_loop(0, ept // 2048, outer, 0)

    pltpu.sync_copy(dacc, sdeg.at[s])
    plsc.subcore_barrier()

    r0 = s * RPT

    def z2(i, _):
        acc_s[pl.ds(i * L, L)] = zero16
        return 0
    lax.fori_loop(0, RPT // L, z2, 0)
    for k in range(NS):
        pltpu.sync_copy(sdeg.at[k, pl.ds(r0, RPT)], tbuf)

        def addk(i, _):
            acc_s[pl.ds(i * L, L)] = acc_s[pl.ds(i * L, L)] + tbuf[pl.ds(i * L, L)]
            return 0
        lax.fori_loop(0, RPT // L, addk, 0)
    pltpu.sync_copy(acc_s, out_hbm.at[c, pl.ds(r0, RPT)])


_deg = pl.kernel(
    _deg_body,
    out_type=jax.ShapeDtypeStruct((NC, NP), jnp.float32),
    mesh=_sc_mesh,
    compiler_params=pltpu.CompilerParams(needs_layout_passes=False),
    scratch_types=[
        pltpu.VMEM((NP,), jnp.float32),
        pltpu.VMEM((2048,), jnp.int32),
        pltpu.VMEM((RPT,), jnp.float32),
        pltpu.VMEM((RPT,), jnp.float32),
        pltpu.VMEM_SHARED((NS, NP), jnp.float32),
    ],
)


def _agg_body(yw_hbm, src_hbm, dst_hbm, out_hbm, acc_sh, sidx, didx, rows, semg, sems):
    c = lax.axis_index("c")
    s = lax.axis_index("s")
    r0 = s * RPT
    pltpu.sync_copy(yw_hbm.at[pl.ds(c * NP + r0, RPT)], acc_sh.at[pl.ds(r0, RPT)])
    plsc.subcore_barrier()

    ebase = s * ERT

    def load_chunk(k):
        slot = lax.rem(k, 2) * CH
        pltpu.sync_copy(src_hbm.at[c, pl.ds(ebase + k * CH, CH)],
                        sidx.at[pl.ds(slot, CH)])
        pltpu.sync_copy(dst_hbm.at[pl.ds(ebase + k * CH, CH)],
                        didx.at[pl.ds(slot, CH)])

    def srow(g):
        return lax.rem(g // CH, 2) * CH + lax.rem(g, CH)

    def issue_gather(g):
        b = lax.rem(g, 2) * 128
        pltpu.async_copy(yw_hbm.at[sidx.at[srow(g)]], rows.at[pl.ds(b, 128)], semg)

    def wait_gather(g):
        b = lax.rem(g, 2) * 128
        pltpu.make_async_copy(yw_hbm.at[sidx.at[srow(g)]],
                              rows.at[pl.ds(b, 128)], semg).wait()

    def issue_scatter(g):
        b = lax.rem(g, 2) * 128
        pltpu.async_copy(rows.at[pl.ds(b, 128)], acc_sh.at[didx.at[srow(g)]],
                         sems, add=True)

    def wait_scatter(g):
        b = lax.rem(g, 2) * 128
        pltpu.make_async_copy(rows.at[pl.ds(b, 128)],
                              acc_sh.at[didx.at[srow(g)]], sems).wait()

    load_chunk(0)
    issue_gather(0)
    issue_gather(1)

    def step(g, _):
        wait_gather(g)
        issue_scatter(g)

        @pl.when(jnp.logical_and(lax.rem(g + 2, CH) == 0, g + 2 < NGRP))
        def _():
            load_chunk((g + 2) // CH)

        wait_scatter(g)

        @pl.when(g + 2 < NGRP)
        def _():
            issue_gather(g + 2)
        return 0
    lax.fori_loop(0, NGRP, step, 0)

    plsc.subcore_barrier()
    pltpu.sync_copy(acc_sh.at[pl.ds(r0, RPT)], out_hbm.at[c, pl.ds(r0, RPT)])


_agg = pl.kernel(
    _agg_body,
    out_type=jax.ShapeDtypeStruct((NC, NP, HC), jnp.float32),
    mesh=_sc_mesh,
    compiler_params=pltpu.CompilerParams(needs_layout_passes=False),
    scratch_types=[
        pltpu.VMEM_SHARED((NP, HC), jnp.float32),
        pltpu.VMEM((2 * CH, 128), jnp.int32),
        pltpu.VMEM((2 * CH, 128), jnp.int32),
        pltpu.VMEM((256, HC), jnp.float32),
        pltpu.SemaphoreType.DMA,
        pltpu.SemaphoreType.DMA,
    ],
)


def _k0mm_body(x_ref, w_ref, o_ref):
    o_ref[...] = jnp.dot(x_ref[...], w_ref[...],
                         preferred_element_type=jnp.float32)


_k0mm = pl.pallas_call(
    _k0mm_body,
    grid=(NB,),
    in_specs=[
        pl.BlockSpec((BR, D), lambda i: (i, 0)),
        pl.BlockSpec((D, H), lambda i: (0, 0)),
    ],
    out_specs=pl.BlockSpec((BR, H), lambda i: (i, 0)),
    out_shape=jax.ShapeDtypeStruct((NP, H), jnp.float32),
)


def _scale_body(t_ref, dv_ref, o_ref):
    yw = dv_ref[...] * t_ref[...]
    o_ref[0] = yw[:, :HC]
    o_ref[1] = yw[:, HC:]


_scale = pl.pallas_call(
    _scale_body,
    grid=(NB,),
    in_specs=[
        pl.BlockSpec((BR, H), lambda i: (i, 0)),
        pl.BlockSpec((BR, 1), lambda i: (i, 0)),
    ],
    out_specs=pl.BlockSpec((2, BR, HC), lambda i: (0, i, 0)),
    out_shape=jax.ShapeDtypeStruct((NC, NP, HC), jnp.float32),
)


def _mid_body(acc_ref, dv_ref, b_ref, w_ref, o_ref):
    a = jnp.concatenate([acc_ref[0], acc_ref[1]], axis=1)
    dv = dv_ref[...]
    h = jnp.maximum(dv * a + b_ref[...], 0.0)
    t = jnp.dot(h, w_ref[...], preferred_element_type=jnp.float32)
    yw = dv * t
    o_ref[0] = yw[:, :HC]
    o_ref[1] = yw[:, HC:]


_mid = pl.pallas_call(
    _mid_body,
    grid=(NB,),
    in_specs=[
        pl.BlockSpec((2, BR, HC), lambda i: (0, i, 0)),
        pl.BlockSpec((BR, 1), lambda i: (i, 0)),
        pl.BlockSpec((1, H), lambda i: (0, 0)),
        pl.BlockSpec((H, H), lambda i: (0, 0)),
    ],
    out_specs=pl.BlockSpec((2, BR, HC), lambda i: (0, i, 0)),
    out_shape=jax.ShapeDtypeStruct((NC, NP, HC), jnp.float32),
)


def _k3_body(acc_ref, dv_ref, b_ref, bt_ref, wl_ref, bl_ref, o_ref, pool_scr, cnt_scr):
    i = pl.program_id(0)

    @pl.when(i == 0)
    def _():
        pool_scr[...] = jnp.zeros_like(pool_scr)
        cnt_scr[...] = jnp.zeros_like(cnt_scr)

    a = jnp.concatenate([acc_ref[0], acc_ref[1]], axis=1)
    h = jnp.maximum(dv_ref[...] * a + b_ref[...], 0.0)
    bt = bt_ref[0]
    oh = (lax.broadcasted_iota(jnp.int32, (G, BR), 0) == bt).astype(jnp.float32)
    pool_scr[...] += jnp.dot(oh, h, preferred_element_type=jnp.float32)
    cnt_scr[...] += jnp.sum(oh, axis=1, keepdims=True)

    @pl.when(i == NB - 1)
    def _():
        mean = pool_scr[...] / jnp.maximum(cnt_scr[...], 1.0)
        logit = jnp.sum(mean * wl_ref[...], axis=1, keepdims=True) + bl_ref[...]
        o_ref[...] = 1.0 / (1.0 + jnp.exp(-logit))


_k3 = pl.pallas_call(
    _k3_body,
    grid=(NB,),
    in_specs=[
        pl.BlockSpec((2, BR, HC), lambda i: (0, i, 0)),
        pl.BlockSpec((BR, 1), lambda i: (i, 0)),
        pl.BlockSpec((1, H), lambda i: (0, 0)),
        pl.BlockSpec((1, 1, BR), lambda i: (i, 0, 0)),
        pl.BlockSpec((1, H), lambda i: (0, 0)),
        pl.BlockSpec((1, 1), lambda i: (0, 0)),
    ],
    out_specs=pl.BlockSpec((G, 1), lambda i: (0, 0)),
    out_shape=jax.ShapeDtypeStruct((G, 1), jnp.float32),
    scratch_shapes=[
        pltpu.VMEM((G, H), jnp.float32),
        pltpu.VMEM((G, 1), jnp.float32),
    ],
)


def kernel(x, edge_index, batch, W1, b1, W2, b2, W3, b3, Wl, bl):
    src = edge_index[0].astype(jnp.int32)
    dst = edge_index[1].astype(jnp.int32)
    pad_src = jnp.full((EP - E,), PAD_SRC, jnp.int32)
    pad_dst = jnp.full((EP - E,), PAD_DST, jnp.int32)
    srcp = jnp.concatenate([src, pad_src]).reshape(ER, 128)
    dstp = jnp.concatenate([dst, pad_dst]).reshape(ER, 128)
    srcpc = jnp.stack([srcp, srcp + NP])
    dst1d = jnp.concatenate([dst, pad_dst])
    xp = jnp.concatenate([x, jnp.zeros((NP - N, D), x.dtype)])
    batchp = jnp.concatenate(
        [batch.astype(jnp.int32), jnp.full((NP - N,), G, jnp.int32)]
    ).reshape(NB, 1, BR)

    degp = _deg(dst1d)
    t1 = _k0mm(xp, W1)
    deg = degp[0] + degp[1] + 1.0
    dinv = lax.rsqrt(jnp.maximum(deg, 1.0))[:, None]

    yw1 = _scale(t1, dinv)
    acc1 = _agg(yw1.reshape(NC * NP, HC), srcpc, dstp)
    yw2 = _mid(acc1, dinv, b1.reshape(1, H), W2)
    acc2 = _agg(yw2.reshape(NC * NP, HC), srcpc, dstp)
    yw3 = _mid(acc2, dinv, b2.reshape(1, H), W3)
    acc3 = _agg(yw3.reshape(NC * NP, HC), srcpc, dstp)
    out = _k3(acc3, dinv, b3.reshape(1, H), batchp, Wl.reshape(1, H), bl.reshape(1, 1))
    return out[:, 0]


# speedup vs baseline: 1.0341x; 1.0341x over previous
import jax
import jax.numpy as jnp
from jax import lax
from jax.experimental import pallas as pl
from jax.experimental.pallas import tpu as pltpu
from jax.experimental.pallas import tpu_sc as plsc

N = 10000
E = 320000
D = 128
H = 256
G = 64
HC = 128
NP = 10240
EP = 327680
ER = EP // 128
NC = 2
NS = 16
L = 16
RPT = NP // NS
ERT = ER // NS
CH = 16
NGRP = ERT
PAD_SRC = 10016
PAD_DST = 10008
NB = 16
BR = NP // NB

_sc_mesh = plsc.VectorSubcoreMesh(
    core_axis_name="c", subcore_axis_name="s", num_cores=NC, num_subcores=NS)


def _deg_body(dst_hbm, out_hbm, dacc, dbuf, acc_s, tbuf, sdeg):
    c = lax.axis_index("c")
    s = lax.axis_index("s")
    zero16 = jnp.zeros((L,), jnp.float32)
    ones16 = jnp.ones((L,), jnp.float32)

    def zloop(i, _):
        dacc[pl.ds(i * L, L)] = zero16
        return 0
    lax.fori_loop(0, NP // L, zloop, 0)

    epc = EP // NC
    ept = epc // NS
    base = c * epc + s * ept

    def outer(g, _):
        pltpu.sync_copy(dst_hbm.at[pl.ds(base + g * 2048, 2048)], dbuf)

        def inner(j, _):
            idx = dbuf[pl.ds(j * L, L)]
            plsc.addupdate_scatter(dacc, [idx], ones16)
            return 0
        lax.fori_loop(0, 2048 // L, inner, 0)
        return 0
    lax.fori_loop(0, ept // 2048, outer, 0)

    pltpu.sync_copy(dacc, sdeg.at[s])
    plsc.subcore_barrier()

    r0 = s * RPT

    def z2(i, _):
        acc_s[pl.ds(i * L, L)] = zero16
        return 0
    lax.fori_loop(0, RPT // L, z2, 0)
    for k in range(NS):
        pltpu.sync_copy(sdeg.at[k, pl.ds(r0, RPT)], tbuf)

        def addk(i, _):
            acc_s[pl.ds(i * L, L)] = acc_s[pl.ds(i * L, L)] + tbuf[pl.ds(i * L, L)]
            return 0
        lax.fori_loop(0, RPT // L, addk, 0)
    pltpu.sync_copy(acc_s, out_hbm.at[c, pl.ds(r0, RPT)])


_deg = pl.kernel(
    _deg_body,
    out_type=jax.ShapeDtypeStruct((NC, NP), jnp.float32),
    mesh=_sc_mesh,
    compiler_params=pltpu.CompilerParams(needs_layout_passes=False),
    scratch_types=[
        pltpu.VMEM((NP,), jnp.float32),
        pltpu.VMEM((2048,), jnp.int32),
        pltpu.VMEM((RPT,), jnp.float32),
        pltpu.VMEM((RPT,), jnp.float32),
        pltpu.VMEM_SHARED((NS, NP), jnp.float32),
    ],
)


def _agg_body(yw_hbm, src_hbm, dst_hbm, out_hbm, acc_sh, sidx, didx, rows, semg, sems, semi):
    c = lax.axis_index("c")
    s = lax.axis_index("s")
    r0 = s * RPT
    ebase = s * ERT

    def load_chunk(k):
        slot = lax.rem(k, 2) * CH
        pltpu.sync_copy(src_hbm.at[c, pl.ds(ebase + k * CH, CH)],
                        sidx.at[pl.ds(slot, CH)])
        pltpu.sync_copy(dst_hbm.at[pl.ds(ebase + k * CH, CH)],
                        didx.at[pl.ds(slot, CH)])

    def issue_load_chunk(k):
        slot = lax.rem(k, 2) * CH
        pltpu.async_copy(src_hbm.at[c, pl.ds(ebase + k * CH, CH)],
                         sidx.at[pl.ds(slot, CH)], semi)
        pltpu.async_copy(dst_hbm.at[pl.ds(ebase + k * CH, CH)],
                         didx.at[pl.ds(slot, CH)], semi)

    def wait_load_chunk(k):
        slot = lax.rem(k, 2) * CH
        pltpu.make_async_copy(src_hbm.at[c, pl.ds(ebase + k * CH, CH)],
                              sidx.at[pl.ds(slot, CH)], semi).wait()
        pltpu.make_async_copy(dst_hbm.at[pl.ds(ebase + k * CH, CH)],
                              didx.at[pl.ds(slot, CH)], semi).wait()

    def srow(g):
        return lax.rem(g // CH, 2) * CH + lax.rem(g, CH)

    def issue_gather(g):
        b = lax.rem(g, 2) * 128
        pltpu.async_copy(yw_hbm.at[sidx.at[srow(g)]], rows.at[pl.ds(b, 128)], semg)

    def wait_gather(g):
        b = lax.rem(g, 2) * 128
        pltpu.make_async_copy(yw_hbm.at[sidx.at[srow(g)]],
                              rows.at[pl.ds(b, 128)], semg).wait()

    def issue_scatter(g):
        b = lax.rem(g, 2) * 128
        pltpu.async_copy(rows.at[pl.ds(b, 128)], acc_sh.at[didx.at[srow(g)]],
                         sems, add=True)

    def wait_scatter(g):
        b = lax.rem(g, 2) * 128
        pltpu.make_async_copy(rows.at[pl.ds(b, 128)],
                              acc_sh.at[didx.at[srow(g)]], sems).wait()

    cpi = pltpu.async_copy(yw_hbm.at[pl.ds(c * NP + r0, RPT)],
                           acc_sh.at[pl.ds(r0, RPT)], semi)
    load_chunk(0)
    issue_gather(0)
    issue_gather(1)
    cpi.wait()
    plsc.subcore_barrier()

    def step(g, _):
        wait_gather(g)
        issue_scatter(g)

        @pl.when(jnp.logical_and(lax.rem(g, CH) == 0, g // CH + 1 < NGRP // CH))
        def _():
            issue_load_chunk(g // CH + 1)

        @pl.when(jnp.logical_and(lax.rem(g + 2, CH) == 0, g + 2 < NGRP))
        def _():
            wait_load_chunk((g + 2) // CH)

        wait_scatter(g)

        @pl.when(g + 2 < NGRP)
        def _():
            issue_gather(g + 2)
        return 0
    lax.fori_loop(0, NGRP, step, 0)

    plsc.subcore_barrier()
    pltpu.sync_copy(acc_sh.at[pl.ds(r0, RPT)], out_hbm.at[c, pl.ds(r0, RPT)])


_agg = pl.kernel(
    _agg_body,
    out_type=jax.ShapeDtypeStruct((NC, NP, HC), jnp.float32),
    mesh=_sc_mesh,
    compiler_params=pltpu.CompilerParams(needs_layout_passes=False),
    scratch_types=[
        pltpu.VMEM_SHARED((NP, HC), jnp.float32),
        pltpu.VMEM((2 * CH, 128), jnp.int32),
        pltpu.VMEM((2 * CH, 128), jnp.int32),
        pltpu.VMEM((256, HC), jnp.float32),
        pltpu.SemaphoreType.DMA,
        pltpu.SemaphoreType.DMA,
        pltpu.SemaphoreType.DMA,
    ],
)


def _k0_body(x_ref, w_ref, dv_ref, o_ref):
    t = jnp.dot(x_ref[...], w_ref[...], preferred_element_type=jnp.float32)
    yw = dv_ref[...] * t
    o_ref[0] = yw[:, :HC]
    o_ref[1] = yw[:, HC:]


_k0 = pl.pallas_call(
    _k0_body,
    grid=(NB,),
    in_specs=[
        pl.BlockSpec((BR, D), lambda i: (i, 0)),
        pl.BlockSpec((D, H), lambda i: (0, 0)),
        pl.BlockSpec((BR, 1), lambda i: (i, 0)),
    ],
    out_specs=pl.BlockSpec((2, BR, HC), lambda i: (0, i, 0)),
    out_shape=jax.ShapeDtypeStruct((NC, NP, HC), jnp.float32),
)


def _mid_body(acc_ref, dv_ref, b_ref, w_ref, o_ref):
    a = jnp.concatenate([acc_ref[0], acc_ref[1]], axis=1)
    dv = dv_ref[...]
    h = jnp.maximum(dv * a + b_ref[...], 0.0)
    t = jnp.dot(h, w_ref[...], preferred_element_type=jnp.float32)
    yw = dv * t
    o_ref[0] = yw[:, :HC]
    o_ref[1] = yw[:, HC:]


_mid = pl.pallas_call(
    _mid_body,
    grid=(NB,),
    in_specs=[
        pl.BlockSpec((2, BR, HC), lambda i: (0, i, 0)),
        pl.BlockSpec((BR, 1), lambda i: (i, 0)),
        pl.BlockSpec((1, H), lambda i: (0, 0)),
        pl.BlockSpec((H, H), lambda i: (0, 0)),
    ],
    out_specs=pl.BlockSpec((2, BR, HC), lambda i: (0, i, 0)),
    out_shape=jax.ShapeDtypeStruct((NC, NP, HC), jnp.float32),
)


def _k3_body(acc_ref, dv_ref, b_ref, bt_ref, wl_ref, bl_ref, o_ref, pool_scr, cnt_scr):
    i = pl.program_id(0)

    @pl.when(i == 0)
    def _():
        pool_scr[...] = jnp.zeros_like(pool_scr)
        cnt_scr[...] = jnp.zeros_like(cnt_scr)

    a = jnp.concatenate([acc_ref[0], acc_ref[1]], axis=1)
    h = jnp.maximum(dv_ref[...] * a + b_ref[...], 0.0)
    bt = bt_ref[0]
    oh = (lax.broadcasted_iota(jnp.int32, (G, BR), 0) == bt).astype(jnp.float32)
    pool_scr[...] += jnp.dot(oh, h, preferred_element_type=jnp.float32)
    cnt_scr[...] += jnp.sum(oh, axis=1, keepdims=True)

    @pl.when(i == NB - 1)
    def _():
        mean = pool_scr[...] / jnp.maximum(cnt_scr[...], 1.0)
        logit = jnp.sum(mean * wl_ref[...], axis=1, keepdims=True) + bl_ref[...]
        o_ref[...] = 1.0 / (1.0 + jnp.exp(-logit))


_k3 = pl.pallas_call(
    _k3_body,
    grid=(NB,),
    in_specs=[
        pl.BlockSpec((2, BR, HC), lambda i: (0, i, 0)),
        pl.BlockSpec((BR, 1), lambda i: (i, 0)),
        pl.BlockSpec((1, H), lambda i: (0, 0)),
        pl.BlockSpec((1, 1, BR), lambda i: (i, 0, 0)),
        pl.BlockSpec((1, H), lambda i: (0, 0)),
        pl.BlockSpec((1, 1), lambda i: (0, 0)),
    ],
    out_specs=pl.BlockSpec((G, 1), lambda i: (0, 0)),
    out_shape=jax.ShapeDtypeStruct((G, 1), jnp.float32),
    scratch_shapes=[
        pltpu.VMEM((G, H), jnp.float32),
        pltpu.VMEM((G, 1), jnp.float32),
    ],
)


def kernel(x, edge_index, batch, W1, b1, W2, b2, W3, b3, Wl, bl):
    src = edge_index[0].astype(jnp.int32)
    dst = edge_index[1].astype(jnp.int32)
    pad_src = jnp.full((EP - E,), PAD_SRC, jnp.int32)
    pad_dst = jnp.full((EP - E,), PAD_DST, jnp.int32)
    srcp = jnp.concatenate([src, pad_src]).reshape(ER, 128)
    dstp = jnp.concatenate([dst, pad_dst]).reshape(ER, 128)
    srcpc = jnp.stack([srcp, srcp + NP])
    dst1d = jnp.concatenate([dst, pad_dst])
    xp = jnp.concatenate([x, jnp.zeros((NP - N, D), x.dtype)])
    batchp = jnp.concatenate(
        [batch.astype(jnp.int32), jnp.full((NP - N,), G, jnp.int32)]
    ).reshape(NB, 1, BR)

    degp = _deg(dst1d)
    deg = degp[0] + degp[1] + 1.0
    dinv = lax.rsqrt(jnp.maximum(deg, 1.0))[:, None]

    yw1 = _k0(xp, W1, dinv)
    acc1 = _agg(yw1.reshape(NC * NP, HC), srcpc, dstp)
    yw2 = _mid(acc1, dinv, b1.reshape(1, H), W2)
    acc2 = _agg(yw2.reshape(NC * NP, HC), srcpc, dstp)
    yw3 = _mid(acc2, dinv, b2.reshape(1, H), W3)
    acc3 = _agg(yw3.reshape(NC * NP, HC), srcpc, dstp)
    out = _k3(acc3, dinv, b3.reshape(1, H), batchp, Wl.reshape(1, H), bl.reshape(1, 1))
    return out[:, 0]
